# Initial kernel scaffold; baseline (speedup 1.0000x reference)
import jax
import jax.numpy as jnp
from jax.experimental import pallas as pl


def kernel(edge_index, edge_type, entity_emb, W1, loop1, b1, W2, loop2, b2):
    raise NotImplementedError("write your pallas kernel here")


# speedup vs baseline: 2.0632x; 2.0632x over previous
import functools

import jax
import jax.numpy as jnp
from jax import lax
from jax.experimental import pallas as pl
from jax.experimental.pallas import tpu as pltpu
from jax.experimental.pallas import tpu_sc as plsc

N_NODES = 10000
NUM_REL = 16
DIM = 128
NUM_EDGES = 320000
SLOTS = NUM_REL + 1

NC = 2
NS = 16
NW = NC * NS
EDGES_PER_W = NUM_EDGES // NW
CHUNK = 80
NCHUNKS = EDGES_PER_W // CHUNK
N_PAD = 10240
ROWS_PER_TILE = N_PAD // NS

BN_MM = 1000
BN_CB = 2000


def _matmul_body(x_ref, w_ref, o_ref):
    o_ref[...] = jnp.dot(x_ref[...], w_ref[...],
                         preferred_element_type=jnp.float32,
                         precision=lax.Precision.HIGHEST)


def _matmul(x, wcat):
    return pl.pallas_call(
        _matmul_body,
        grid=(N_NODES // BN_MM,),
        in_specs=[
            pl.BlockSpec((BN_MM, DIM), lambda i: (i, 0)),
            pl.BlockSpec((DIM, SLOTS * DIM), lambda i: (0, 0)),
        ],
        out_specs=pl.BlockSpec((BN_MM, SLOTS * DIM), lambda i: (i, 0)),
        out_shape=jax.ShapeDtypeStruct((N_NODES, SLOTS * DIM), jnp.float32),
    )(x, wcat)


def _combine_body(p0_ref, p1_ref, lp_ref, b_ref, o_ref):
    acc = p0_ref[...] + p1_ref[...] + lp_ref[...] + b_ref[...]
    o_ref[...] = jnp.maximum(acc, 0.0)


def _combine(p0, p1, x_all, b):
    return pl.pallas_call(
        _combine_body,
        grid=(N_NODES // BN_CB,),
        in_specs=[
            pl.BlockSpec((BN_CB, DIM), lambda i: (i, 0)),
            pl.BlockSpec((BN_CB, DIM), lambda i: (i, 0)),
            pl.BlockSpec((BN_CB, DIM), lambda i: (i, NUM_REL)),
            pl.BlockSpec((1, DIM), lambda i: (0, 0)),
        ],
        out_specs=pl.BlockSpec((BN_CB, DIM), lambda i: (i, 0)),
        out_shape=jax.ShapeDtypeStruct((N_NODES, DIM), jnp.float32),
    )(p0, p1, x_all, b.reshape(1, DIM))


def _sc_body(xall_hbm, g_hbm, dst_hbm, zeros_hbm, out_hbm,
             gbuf, dbuf, rows, acc, sem):
    c = lax.axis_index("c")
    s = lax.axis_index("s")
    wid = s * NC + c

    row0 = s * ROWS_PER_TILE
    pltpu.sync_copy(zeros_hbm.at[pl.ds(row0, ROWS_PER_TILE)],
                    acc.at[pl.ds(row0, ROWS_PER_TILE)])
    plsc.subcore_barrier()

    base = wid * EDGES_PER_W

    def chunk_body(i, carry):
        off = base + i * CHUNK
        pltpu.sync_copy(g_hbm.at[pl.ds(off, CHUNK)], gbuf)
        pltpu.sync_copy(dst_hbm.at[pl.ds(off, CHUNK)], dbuf)
        pltpu.async_copy(xall_hbm.at[gbuf], rows, sem).wait()
        pltpu.sync_copy(rows, acc.at[dbuf], add=True)
        return carry

    lax.fori_loop(0, NCHUNKS, chunk_body, 0)
    plsc.subcore_barrier()

    pltpu.sync_copy(acc.at[pl.ds(row0, ROWS_PER_TILE)],
                    out_hbm.at[c, pl.ds(row0, ROWS_PER_TILE)])


@functools.lru_cache(maxsize=None)
def _build_sc_scatter():
    return pl.kernel(
        _sc_body,
        out_type=jax.ShapeDtypeStruct((NC, N_PAD, DIM), jnp.float32),
        mesh=plsc.VectorSubcoreMesh(core_axis_name="c", subcore_axis_name="s"),
        scratch_types=[
            pltpu.VMEM((CHUNK,), jnp.int32),
            pltpu.VMEM((CHUNK,), jnp.int32),
            pltpu.VMEM((CHUNK, DIM), jnp.float32),
            pltpu.VMEM_SHARED((N_PAD, DIM), jnp.float32),
            pltpu.SemaphoreType.DMA,
        ],
    )


def _sc_scatter(x_all_flat, g, dst, zeros):
    return _build_sc_scatter()(x_all_flat, g, dst, zeros)


def kernel(edge_index, edge_type, entity_emb, W1, loop1, b1, W2, loop2, b2):
    src = edge_index[0]
    dst = edge_index[1]
    g = src * SLOTS + edge_type
    zeros = jnp.zeros((N_PAD, DIM), jnp.float32)

    x = entity_emb
    for W, lw, b in ((W1, loop1, b1), (W2, loop2, b2)):
        wcat = jnp.concatenate([W, lw[None]], axis=0)
        wcat = wcat.transpose(1, 0, 2).reshape(DIM, SLOTS * DIM)
        x_all = _matmul(x, wcat)
        part = _sc_scatter(x_all.reshape(N_NODES * SLOTS, DIM), g, dst, zeros)
        x = _combine(part[0], part[1], x_all, b)
    return x


# speedup vs baseline: 4.2970x; 2.0827x over previous
import functools

import jax
import jax.numpy as jnp
from jax import lax
from jax.experimental import pallas as pl
from jax.experimental.pallas import tpu as pltpu
from jax.experimental.pallas import tpu_sc as plsc

N_NODES = 10000
NUM_REL = 16
DIM = 128
NUM_EDGES = 320000
SLOTS = NUM_REL + 1

NC = 2
NS = 16
NW = NC * NS
EDGES_PER_W = NUM_EDGES // NW
CHUNK = 128
NCHUNKS = -(-EDGES_PER_W // CHUNK)
PAD_W = NCHUNKS * CHUNK - EDGES_PER_W
N_PAD = 10240
ROWS_PER_TILE = N_PAD // NS

BN_MM = 1000
BN_CB = 2000


def _matmul_body(x_ref, w_ref, o_ref):
    res = jnp.dot(x_ref[...], w_ref[...],
                  preferred_element_type=jnp.float32)
    for r in range(SLOTS):
        o_ref[r] = res[:, r * DIM:(r + 1) * DIM]


def _matmul(x, wcat):
    return pl.pallas_call(
        _matmul_body,
        grid=(N_NODES // BN_MM,),
        in_specs=[
            pl.BlockSpec((BN_MM, DIM), lambda i: (i, 0)),
            pl.BlockSpec((DIM, SLOTS * DIM), lambda i: (0, 0)),
        ],
        out_specs=pl.BlockSpec((SLOTS, BN_MM, DIM), lambda i: (0, i, 0)),
        out_shape=jax.ShapeDtypeStruct((SLOTS, N_NODES, DIM), jnp.float32),
    )(x, wcat)


def _combine_body(p0_ref, p1_ref, lp_ref, b_ref, o_ref):
    acc = p0_ref[...] + p1_ref[...] + lp_ref[...] + b_ref[...]
    o_ref[...] = jnp.maximum(acc, 0.0)


def _combine(p0, p1, selfloop, b):
    return pl.pallas_call(
        _combine_body,
        grid=(N_NODES // BN_CB,),
        in_specs=[
            pl.BlockSpec((BN_CB, DIM), lambda i: (i, 0)),
            pl.BlockSpec((BN_CB, DIM), lambda i: (i, 0)),
            pl.BlockSpec((BN_CB, DIM), lambda i: (i, 0)),
            pl.BlockSpec((1, DIM), lambda i: (0, 0)),
        ],
        out_specs=pl.BlockSpec((BN_CB, DIM), lambda i: (i, 0)),
        out_shape=jax.ShapeDtypeStruct((N_NODES, DIM), jnp.float32),
    )(p0, p1, selfloop, b.reshape(1, DIM))


def _sc_body(xall_hbm, g_hbm, dst_hbm, zeros_hbm, out_hbm,
             idx, rows, acc, sem0):
    sem1 = sem0
    gidx = idx.at[0]
    didx = idx.at[1]
    rows0 = rows.at[0]
    rows1 = rows.at[1]
    c = lax.axis_index("c")
    s = lax.axis_index("s")
    wid = s * NC + c

    row0 = s * ROWS_PER_TILE
    pltpu.sync_copy(zeros_hbm.at[pl.ds(row0, ROWS_PER_TILE)],
                    acc.at[pl.ds(row0, ROWS_PER_TILE)])
    pltpu.sync_copy(g_hbm.at[wid], gidx)
    pltpu.sync_copy(dst_hbm.at[wid], didx)
    plsc.subcore_barrier()

    pltpu.async_copy(xall_hbm.at[gidx.at[0]], rows0, sem0)

    def pair_body(p, carry):
        i0 = p * 2
        pltpu.make_async_copy(xall_hbm.at[gidx.at[i0]], rows0, sem0).wait()
        pltpu.async_copy(xall_hbm.at[gidx.at[i0 + 1]], rows1, sem1)
        pltpu.sync_copy(rows0, acc.at[didx.at[i0]], add=True)
        pltpu.make_async_copy(xall_hbm.at[gidx.at[i0 + 1]], rows1, sem1).wait()
        pltpu.async_copy(xall_hbm.at[gidx.at[i0 + 2]], rows0, sem0)
        pltpu.sync_copy(rows1, acc.at[didx.at[i0 + 1]], add=True)
        return carry

    lax.fori_loop(0, NCHUNKS // 2, pair_body, 0)

    last = NCHUNKS - 1
    pltpu.make_async_copy(xall_hbm.at[gidx.at[last]], rows0, sem0).wait()
    pltpu.sync_copy(rows0, acc.at[didx.at[last]], add=True)
    plsc.subcore_barrier()

    pltpu.sync_copy(acc.at[pl.ds(row0, ROWS_PER_TILE)],
                    out_hbm.at[c, pl.ds(row0, ROWS_PER_TILE)])


@functools.lru_cache(maxsize=None)
def _build_sc_scatter():
    return pl.kernel(
        _sc_body,
        out_type=jax.ShapeDtypeStruct((NC, N_PAD, DIM), jnp.float32),
        mesh=plsc.VectorSubcoreMesh(core_axis_name="c", subcore_axis_name="s"),
        scratch_types=[
            pltpu.VMEM((2, NCHUNKS, CHUNK), jnp.int32),
            pltpu.VMEM((2, CHUNK, DIM), jnp.float32),
            pltpu.VMEM_SHARED((N_PAD, DIM), jnp.float32),
            pltpu.SemaphoreType.DMA,
        ],
    )


def _sc_scatter(x_all_flat, g3, d3, zeros):
    return _build_sc_scatter()(x_all_flat, g3, d3, zeros)


def kernel(edge_index, edge_type, entity_emb, W1, loop1, b1, W2, loop2, b2):
    src = edge_index[0]
    dst = edge_index[1]
    g = (edge_type * N_NODES + src).reshape(NW, EDGES_PER_W)
    d = dst.reshape(NW, EDGES_PER_W)
    pad_g = jnp.broadcast_to(jnp.arange(PAD_W, dtype=jnp.int32), (NW, PAD_W))
    pad_d = pad_g + N_NODES
    g3 = jnp.concatenate([g, pad_g], axis=1).reshape(NW, NCHUNKS, CHUNK)
    d3 = jnp.concatenate([d, pad_d], axis=1).reshape(NW, NCHUNKS, CHUNK)
    zeros = jnp.zeros((N_PAD, DIM), jnp.float32)

    x = entity_emb
    for W, lw, b in ((W1, loop1, b1), (W2, loop2, b2)):
        wcat = jnp.concatenate([W, lw[None]], axis=0)
        wcat = wcat.transpose(1, 0, 2).reshape(DIM, SLOTS * DIM)
        x_all = _matmul(x, wcat)
        part = _sc_scatter(x_all.reshape(SLOTS * N_NODES, DIM), g3, d3, zeros)
        x = _combine(part[0], part[1], x_all[NUM_REL], b)
    return x


# speedup vs baseline: 4.3330x; 1.0084x over previous
import functools

import jax
import jax.numpy as jnp
from jax import lax
from jax.experimental import pallas as pl
from jax.experimental.pallas import tpu as pltpu
from jax.experimental.pallas import tpu_sc as plsc

N_NODES = 10000
NUM_REL = 16
DIM = 128
NUM_EDGES = 320000
SLOTS = NUM_REL + 1

NC = 2
NS = 16
NW = NC * NS
EDGES_PER_W = NUM_EDGES // NW
CHUNK = 128
NCHUNKS = -(-EDGES_PER_W // CHUNK)
PAD_W = NCHUNKS * CHUNK - EDGES_PER_W
N_PAD = 10240
ROWS_PER_TILE = N_PAD // NS

BN_MM = 1000
BN_CB = 2000


def _matmul_body(x_ref, w_ref, o_ref):
    res = jnp.dot(x_ref[...], w_ref[...],
                  preferred_element_type=jnp.float32)
    for r in range(SLOTS):
        o_ref[r] = res[:, r * DIM:(r + 1) * DIM]


def _matmul(x, wcat):
    return pl.pallas_call(
        _matmul_body,
        grid=(N_NODES // BN_MM,),
        in_specs=[
            pl.BlockSpec((BN_MM, DIM), lambda i: (i, 0)),
            pl.BlockSpec((DIM, SLOTS * DIM), lambda i: (0, 0)),
        ],
        out_specs=pl.BlockSpec((SLOTS, BN_MM, DIM), lambda i: (0, i, 0)),
        out_shape=jax.ShapeDtypeStruct((SLOTS, N_NODES, DIM), jnp.float32),
    )(x, wcat)


def _matmul_fused_body(p0_ref, p1_ref, lp_ref, b_ref, w_ref, o_ref):
    x = jnp.maximum(p0_ref[...] + p1_ref[...] + lp_ref[...] + b_ref[...], 0.0)
    res = jnp.dot(x, w_ref[...], preferred_element_type=jnp.float32)
    for r in range(SLOTS):
        o_ref[r] = res[:, r * DIM:(r + 1) * DIM]


def _matmul_fused(p0, p1, selfloop, b, wcat):
    return pl.pallas_call(
        _matmul_fused_body,
        grid=(N_NODES // BN_MM,),
        in_specs=[
            pl.BlockSpec((BN_MM, DIM), lambda i: (i, 0)),
            pl.BlockSpec((BN_MM, DIM), lambda i: (i, 0)),
            pl.BlockSpec((BN_MM, DIM), lambda i: (i, 0)),
            pl.BlockSpec((1, DIM), lambda i: (0, 0)),
            pl.BlockSpec((DIM, SLOTS * DIM), lambda i: (0, 0)),
        ],
        out_specs=pl.BlockSpec((SLOTS, BN_MM, DIM), lambda i: (0, i, 0)),
        out_shape=jax.ShapeDtypeStruct((SLOTS, N_NODES, DIM), jnp.float32),
    )(p0, p1, selfloop, b.reshape(1, DIM), wcat)


def _combine_body(p0_ref, p1_ref, lp_ref, b_ref, o_ref):
    acc = p0_ref[...] + p1_ref[...] + lp_ref[...] + b_ref[...]
    o_ref[...] = jnp.maximum(acc, 0.0)


def _combine(p0, p1, selfloop, b):
    return pl.pallas_call(
        _combine_body,
        grid=(N_NODES // BN_CB,),
        in_specs=[
            pl.BlockSpec((BN_CB, DIM), lambda i: (i, 0)),
            pl.BlockSpec((BN_CB, DIM), lambda i: (i, 0)),
            pl.BlockSpec((BN_CB, DIM), lambda i: (i, 0)),
            pl.BlockSpec((1, DIM), lambda i: (0, 0)),
        ],
        out_specs=pl.BlockSpec((BN_CB, DIM), lambda i: (i, 0)),
        out_shape=jax.ShapeDtypeStruct((N_NODES, DIM), jnp.float32),
    )(p0, p1, selfloop, b.reshape(1, DIM))


def _sc_body(xall_hbm, g_hbm, dst_hbm, zeros_hbm, out_hbm,
             idx, rows, acc, sem0):
    sem1 = sem0
    gidx = idx.at[0]
    didx = idx.at[1]
    rows0 = rows.at[0]
    rows1 = rows.at[1]
    c = lax.axis_index("c")
    s = lax.axis_index("s")
    wid = s * NC + c

    row0 = s * ROWS_PER_TILE
    pltpu.sync_copy(zeros_hbm.at[pl.ds(row0, ROWS_PER_TILE)],
                    acc.at[pl.ds(row0, ROWS_PER_TILE)])
    pltpu.sync_copy(g_hbm.at[wid], gidx)
    pltpu.sync_copy(dst_hbm.at[wid], didx)
    plsc.subcore_barrier()

    pltpu.async_copy(xall_hbm.at[gidx.at[0]], rows0, sem0)

    def pair_body(p, carry):
        i0 = p * 2
        pltpu.make_async_copy(xall_hbm.at[gidx.at[i0]], rows0, sem0).wait()
        pltpu.async_copy(xall_hbm.at[gidx.at[i0 + 1]], rows1, sem1)
        pltpu.sync_copy(rows0, acc.at[didx.at[i0]], add=True)
        pltpu.make_async_copy(xall_hbm.at[gidx.at[i0 + 1]], rows1, sem1).wait()
        pltpu.async_copy(xall_hbm.at[gidx.at[i0 + 2]], rows0, sem0)
        pltpu.sync_copy(rows1, acc.at[didx.at[i0 + 1]], add=True)
        return carry

    lax.fori_loop(0, NCHUNKS // 2, pair_body, 0)

    last = NCHUNKS - 1
    pltpu.make_async_copy(xall_hbm.at[gidx.at[last]], rows0, sem0).wait()
    pltpu.sync_copy(rows0, acc.at[didx.at[last]], add=True)
    plsc.subcore_barrier()

    pltpu.sync_copy(acc.at[pl.ds(row0, ROWS_PER_TILE)],
                    out_hbm.at[c, pl.ds(row0, ROWS_PER_TILE)])


@functools.lru_cache(maxsize=None)
def _build_sc_scatter():
    return pl.kernel(
        _sc_body,
        out_type=jax.ShapeDtypeStruct((NC, N_PAD, DIM), jnp.float32),
        mesh=plsc.VectorSubcoreMesh(core_axis_name="c", subcore_axis_name="s"),
        scratch_types=[
            pltpu.VMEM((2, NCHUNKS, CHUNK), jnp.int32),
            pltpu.VMEM((2, CHUNK, DIM), jnp.float32),
            pltpu.VMEM_SHARED((N_PAD, DIM), jnp.float32),
            pltpu.SemaphoreType.DMA,
        ],
    )


def _sc_scatter(x_all_flat, g3, d3, zeros):
    return _build_sc_scatter()(x_all_flat, g3, d3, zeros)


def kernel(edge_index, edge_type, entity_emb, W1, loop1, b1, W2, loop2, b2):
    src = edge_index[0]
    dst = edge_index[1]
    g = (edge_type * N_NODES + src).reshape(NW, EDGES_PER_W)
    d = dst.reshape(NW, EDGES_PER_W)
    pad_g = jnp.broadcast_to(jnp.arange(PAD_W, dtype=jnp.int32), (NW, PAD_W))
    pad_d = pad_g + N_NODES
    g3 = jnp.concatenate([g, pad_g], axis=1).reshape(NW, NCHUNKS, CHUNK)
    d3 = jnp.concatenate([d, pad_d], axis=1).reshape(NW, NCHUNKS, CHUNK)
    zeros = jnp.zeros((N_PAD, DIM), jnp.float32)

    def _wcat(W, lw):
        w = jnp.concatenate([W, lw[None]], axis=0)
        return w.transpose(1, 0, 2).reshape(DIM, SLOTS * DIM)

    x_all = _matmul(entity_emb, _wcat(W1, loop1))
    part = _sc_scatter(x_all.reshape(SLOTS * N_NODES, DIM), g3, d3, zeros)
    x_all2 = _matmul_fused(part[0], part[1], x_all[NUM_REL], b1, _wcat(W2, loop2))
    part2 = _sc_scatter(x_all2.reshape(SLOTS * N_NODES, DIM), g3, d3, zeros)
    return _combine(part2[0], part2[1], x_all2[NUM_REL], b2)


# speedup vs baseline: 4.3418x; 1.0020x over previous
import functools

import jax
import jax.numpy as jnp
from jax import lax
from jax.experimental import pallas as pl
from jax.experimental.pallas import tpu as pltpu
from jax.experimental.pallas import tpu_sc as plsc

N_NODES = 10000
NUM_REL = 16
DIM = 128
NUM_EDGES = 320000
SLOTS = NUM_REL + 1

NC = 2
NS = 16
NW = NC * NS
EDGES_PER_W = NUM_EDGES // NW
CHUNK = 128
NCHUNKS = -(-EDGES_PER_W // CHUNK)
PAD_W = NCHUNKS * CHUNK - EDGES_PER_W
N_PAD = 10240
ROWS_PER_TILE = N_PAD // NS

BN_MM = 2000
BN_CB = 10000


def _matmul_body(x_ref, w_ref, o_ref):
    res = jnp.dot(x_ref[...], w_ref[...],
                  preferred_element_type=jnp.float32)
    for r in range(SLOTS):
        o_ref[r] = res[:, r * DIM:(r + 1) * DIM]


def _matmul(x, wcat):
    return pl.pallas_call(
        _matmul_body,
        grid=(N_NODES // BN_MM,),
        in_specs=[
            pl.BlockSpec((BN_MM, DIM), lambda i: (i, 0)),
            pl.BlockSpec((DIM, SLOTS * DIM), lambda i: (0, 0)),
        ],
        out_specs=pl.BlockSpec((SLOTS, BN_MM, DIM), lambda i: (0, i, 0)),
        out_shape=jax.ShapeDtypeStruct((SLOTS, N_NODES, DIM), jnp.float32),
    )(x, wcat)


def _matmul_fused_body(p0_ref, p1_ref, lp_ref, b_ref, w_ref, o_ref):
    x = jnp.maximum(p0_ref[...] + p1_ref[...] + lp_ref[...] + b_ref[...], 0.0)
    res = jnp.dot(x, w_ref[...], preferred_element_type=jnp.float32)
    for r in range(SLOTS):
        o_ref[r] = res[:, r * DIM:(r + 1) * DIM]


def _matmul_fused(p0, p1, selfloop, b, wcat):
    return pl.pallas_call(
        _matmul_fused_body,
        grid=(N_NODES // BN_MM,),
        in_specs=[
            pl.BlockSpec((BN_MM, DIM), lambda i: (i, 0)),
            pl.BlockSpec((BN_MM, DIM), lambda i: (i, 0)),
            pl.BlockSpec((BN_MM, DIM), lambda i: (i, 0)),
            pl.BlockSpec((1, DIM), lambda i: (0, 0)),
            pl.BlockSpec((DIM, SLOTS * DIM), lambda i: (0, 0)),
        ],
        out_specs=pl.BlockSpec((SLOTS, BN_MM, DIM), lambda i: (0, i, 0)),
        out_shape=jax.ShapeDtypeStruct((SLOTS, N_NODES, DIM), jnp.float32),
    )(p0, p1, selfloop, b.reshape(1, DIM), wcat)


def _combine_body(p0_ref, p1_ref, lp_ref, b_ref, o_ref):
    acc = p0_ref[...] + p1_ref[...] + lp_ref[...] + b_ref[...]
    o_ref[...] = jnp.maximum(acc, 0.0)


def _combine(p0, p1, selfloop, b):
    return pl.pallas_call(
        _combine_body,
        grid=(N_NODES // BN_CB,),
        in_specs=[
            pl.BlockSpec((BN_CB, DIM), lambda i: (i, 0)),
            pl.BlockSpec((BN_CB, DIM), lambda i: (i, 0)),
            pl.BlockSpec((BN_CB, DIM), lambda i: (i, 0)),
            pl.BlockSpec((1, DIM), lambda i: (0, 0)),
        ],
        out_specs=pl.BlockSpec((BN_CB, DIM), lambda i: (i, 0)),
        out_shape=jax.ShapeDtypeStruct((N_NODES, DIM), jnp.float32),
    )(p0, p1, selfloop, b.reshape(1, DIM))


def _sc_body(xall_hbm, g_hbm, dst_hbm, zeros_hbm, out_hbm,
             idx, rows, acc, sem0):
    sem1 = sem0
    gidx = idx.at[0]
    didx = idx.at[1]
    rows0 = rows.at[0]
    rows1 = rows.at[1]
    c = lax.axis_index("c")
    s = lax.axis_index("s")
    wid = s * NC + c

    row0 = s * ROWS_PER_TILE
    pltpu.sync_copy(zeros_hbm.at[pl.ds(row0, ROWS_PER_TILE)],
                    acc.at[pl.ds(row0, ROWS_PER_TILE)])
    pltpu.sync_copy(g_hbm.at[wid], gidx)
    pltpu.sync_copy(dst_hbm.at[wid], didx)
    plsc.subcore_barrier()

    pltpu.async_copy(xall_hbm.at[gidx.at[0]], rows0, sem0)

    def pair_body(p, carry):
        i0 = p * 2
        pltpu.make_async_copy(xall_hbm.at[gidx.at[i0]], rows0, sem0).wait()
        pltpu.async_copy(xall_hbm.at[gidx.at[i0 + 1]], rows1, sem1)
        pltpu.sync_copy(rows0, acc.at[didx.at[i0]], add=True)
        pltpu.make_async_copy(xall_hbm.at[gidx.at[i0 + 1]], rows1, sem1).wait()
        pltpu.async_copy(xall_hbm.at[gidx.at[i0 + 2]], rows0, sem0)
        pltpu.sync_copy(rows1, acc.at[didx.at[i0 + 1]], add=True)
        return carry

    lax.fori_loop(0, NCHUNKS // 2, pair_body, 0)

    last = NCHUNKS - 1
    pltpu.make_async_copy(xall_hbm.at[gidx.at[last]], rows0, sem0).wait()
    pltpu.sync_copy(rows0, acc.at[didx.at[last]], add=True)
    plsc.subcore_barrier()

    pltpu.sync_copy(acc.at[pl.ds(row0, ROWS_PER_TILE)],
                    out_hbm.at[c, pl.ds(row0, ROWS_PER_TILE)])


@functools.lru_cache(maxsize=None)
def _build_sc_scatter():
    return pl.kernel(
        _sc_body,
        out_type=jax.ShapeDtypeStruct((NC, N_PAD, DIM), jnp.float32),
        mesh=plsc.VectorSubcoreMesh(core_axis_name="c", subcore_axis_name="s"),
        scratch_types=[
            pltpu.VMEM((2, NCHUNKS, CHUNK), jnp.int32),
            pltpu.VMEM((2, CHUNK, DIM), jnp.float32),
            pltpu.VMEM_SHARED((N_PAD, DIM), jnp.float32),
            pltpu.SemaphoreType.DMA,
        ],
    )


def _sc_scatter(x_all_flat, g3, d3, zeros):
    return _build_sc_scatter()(x_all_flat, g3, d3, zeros)


def kernel(edge_index, edge_type, entity_emb, W1, loop1, b1, W2, loop2, b2):
    src = edge_index[0]
    dst = edge_index[1]
    g = (edge_type * N_NODES + src).reshape(NW, EDGES_PER_W)
    d = dst.reshape(NW, EDGES_PER_W)
    pad_g = jnp.broadcast_to(jnp.arange(PAD_W, dtype=jnp.int32), (NW, PAD_W))
    pad_d = pad_g + N_NODES
    g3 = jnp.concatenate([g, pad_g], axis=1).reshape(NW, NCHUNKS, CHUNK)
    d3 = jnp.concatenate([d, pad_d], axis=1).reshape(NW, NCHUNKS, CHUNK)
    zeros = jnp.zeros((N_PAD, DIM), jnp.float32)

    def _wcat(W, lw):
        w = jnp.concatenate([W, lw[None]], axis=0)
        return w.transpose(1, 0, 2).reshape(DIM, SLOTS * DIM)

    x_all = _matmul(entity_emb, _wcat(W1, loop1))
    part = _sc_scatter(x_all.reshape(SLOTS * N_NODES, DIM), g3, d3, zeros)
    x_all2 = _matmul_fused(part[0], part[1], x_all[NUM_REL], b1, _wcat(W2, loop2))
    part2 = _sc_scatter(x_all2.reshape(SLOTS * N_NODES, DIM), g3, d3, zeros)
    return _combine(part2[0], part2[1], x_all2[NUM_REL], b2)
